# Initial kernel scaffold; baseline (speedup 1.0000x reference)
import jax
import jax.numpy as jnp
from jax.experimental import pallas as pl


def kernel(x, edge_index, pos_edge_index, neg_edge_index, graph_idx, W1, b1, W2, b2, Wfc1, bfc1, Wfc2, bfc2, Wfc3, bfc3, Wfc4, bfc4):
    raise NotImplementedError("write your pallas kernel here")


# speedup vs baseline: 7.5888x; 7.5888x over previous
import functools

import jax
import jax.numpy as jnp
from jax import lax
from jax.experimental import pallas as pl
from jax.experimental.pallas import tpu as pltpu
from jax.experimental.pallas import tpu_sc as plsc

N = 10000
E = 320000
DE = 320000
C = 128
H = 64

_NC, _NS, _L = 2, 16, 16
_NW = _NC * _NS
_EPT = E // _NW
_B = 80
_NB = _EPT // _B
_NP = 10240
_RP = _NP // _NS
_ZC = 128


def _sc_mesh():
    return plsc.VectorSubcoreMesh(
        core_axis_name="c", subcore_axis_name="s",
        num_cores=_NC, num_subcores=_NS)


def _zero_rows(buf, nrows, ncol16):
    def fill(i, _):
        for l in range(ncol16):
            buf[i, pl.ds(l * _L, _L)] = jnp.zeros((_L,), jnp.float32)
        return 0
    lax.fori_loop(0, nrows, fill, 0)


def _sc_degree(dst3):
    def body(dst_hbm, out_hbm, idx_v, ones_v, zb_v, sh):
        c = lax.axis_index("c")
        s = lax.axis_index("s")
        wid = c * _NS + s
        def fill(i, _):
            ones_v[i, pl.ds(0, _L)] = jnp.ones((_L,), jnp.float32)
            return 0
        lax.fori_loop(0, _B, fill, 0)
        _zero_rows(zb_v, _ZC, 1)
        for k in range(_RP // _ZC):
            pltpu.sync_copy(zb_v, sh.at[pl.ds(s * _RP + k * _ZC, _ZC)])
        plsc.subcore_barrier()
        pltpu.sync_copy(dst_hbm.at[wid], idx_v)
        def step(j, _):
            pltpu.sync_copy(ones_v, sh.at[idx_v.at[j]], add=True)
            return 0
        lax.fori_loop(0, _NB, step, 0)
        plsc.subcore_barrier()
        pltpu.sync_copy(sh.at[pl.ds(s * _RP, _RP)],
                        out_hbm.at[c, pl.ds(s * _RP, _RP)])

    return pl.kernel(
        body,
        out_type=jax.ShapeDtypeStruct((_NC, _NP, _L), jnp.float32),
        mesh=_sc_mesh(),
        compiler_params=pltpu.CompilerParams(use_tc_tiling_on_sc=False),
        scratch_types=[
            pltpu.VMEM((_NB, _B), jnp.int32),
            pltpu.VMEM((_B, _L), jnp.float32),
            pltpu.VMEM((_ZC, _L), jnp.float32),
            pltpu.VMEM_SHARED((_NP, _L), jnp.float32),
        ],
    )(dst3)


def _sc_agg(u, src3, dst3):
    def body(u_hbm, src_hbm, dst_hbm, out_hbm, sidx, didx, rows, zb, sh, sem):
        c = lax.axis_index("c")
        s = lax.axis_index("s")
        wid = c * _NS + s
        _zero_rows(zb, _ZC, H // _L)
        for k in range(_RP // _ZC):
            pltpu.sync_copy(zb, sh.at[pl.ds(s * _RP + k * _ZC, _ZC)])
        plsc.subcore_barrier()
        pltpu.sync_copy(src_hbm.at[wid], sidx)
        pltpu.sync_copy(dst_hbm.at[wid], didx)
        def step(j, _):
            pltpu.async_copy(u_hbm.at[sidx.at[j]], rows, sem).wait()
            pltpu.sync_copy(rows, sh.at[didx.at[j]], add=True)
            return 0
        lax.fori_loop(0, _NB, step, 0)
        plsc.subcore_barrier()
        pltpu.sync_copy(sh.at[pl.ds(s * _RP, _RP)],
                        out_hbm.at[c, pl.ds(s * _RP, _RP)])

    return pl.kernel(
        body,
        out_type=jax.ShapeDtypeStruct((_NC, _NP, H), jnp.float32),
        mesh=_sc_mesh(),
        compiler_params=pltpu.CompilerParams(use_tc_tiling_on_sc=False),
        scratch_types=[
            pltpu.VMEM((_NB, _B), jnp.int32),
            pltpu.VMEM((_NB, _B), jnp.int32),
            pltpu.VMEM((_B, H), jnp.float32),
            pltpu.VMEM((_ZC, H), jnp.float32),
            pltpu.VMEM_SHARED((_NP, H), jnp.float32),
            pltpu.SemaphoreType.DMA,
        ],
    )(u, src3, dst3)


def _sc_decode(za, zb, a3, b3):
    def body(za_hbm, zb_hbm, a_hbm, b_hbm, out_hbm, av, bv, ra, rb, sem):
        c = lax.axis_index("c")
        s = lax.axis_index("s")
        wid = c * _NS + s
        pltpu.sync_copy(a_hbm.at[wid], av)
        pltpu.sync_copy(b_hbm.at[wid], bv)
        def step(j, _):
            cpa = pltpu.async_copy(za_hbm.at[av.at[j]], ra, sem)
            cpb = pltpu.async_copy(zb_hbm.at[bv.at[j]], rb, sem)
            cpa.wait()
            cpb.wait()
            def comp(i, _):
                for l in range(C // _L):
                    sl = pl.ds(l * _L, _L)
                    ra[i, sl] = jnp.maximum(ra[i, sl] + rb[i, sl], 0.0)
                return 0
            lax.fori_loop(0, _B, comp, 0)
            pltpu.sync_copy(ra, out_hbm.at[pl.ds(wid * _EPT + j * _B, _B)])
            return 0
        lax.fori_loop(0, _NB, step, 0)

    return pl.kernel(
        body,
        out_type=jax.ShapeDtypeStruct((DE, C), jnp.float32),
        mesh=_sc_mesh(),
        scratch_types=[
            pltpu.VMEM((_NB, _B), jnp.int32),
            pltpu.VMEM((_NB, _B), jnp.int32),
            pltpu.VMEM((_B, C), jnp.float32),
            pltpu.VMEM((_B, C), jnp.float32),
            pltpu.SemaphoreType.DMA,
        ],
    )(za, zb, a3, b3)


_R = 2000


def _dinv_block(dp):
    deg = dp[0, :, 0:1] + dp[1, :, 0:1] + 1.0
    return lax.rsqrt(deg)


def _tc_encode1(x, W1, degp):
    def body(x_ref, w_ref, dp_ref, u0_ref, u1_ref, u2_ref, u3_ref):
        dinv = _dinv_block(dp_ref[...])
        xw = jnp.dot(x_ref[...], w_ref[...], preferred_element_type=jnp.float32)
        u = xw * dinv
        u0_ref[...] = u[:, 0 * H:1 * H]
        u1_ref[...] = u[:, 1 * H:2 * H]
        u2_ref[...] = u[:, 2 * H:3 * H]
        u3_ref[...] = u[:, 3 * H:4 * H]
    return pl.pallas_call(
        body,
        grid=(N // _R,),
        in_specs=[
            pl.BlockSpec((_R, C), lambda i: (i, 0)),
            pl.BlockSpec((C, 2 * C), lambda i: (0, 0)),
            pl.BlockSpec((_NC, _R, _L), lambda i: (0, i, 0)),
        ],
        out_specs=[pl.BlockSpec((_R, H), lambda i: (i, 0))] * 4,
        out_shape=[jax.ShapeDtypeStruct((N, H), jnp.float32)] * 4,
    )(x, W1, degp)


def _tc_encode2(aggs, us, degp, W2, b1):
    def body(a0, a1, a2, a3, u0, u1, u2, u3, dp_ref, w_ref, b_ref,
             o0_ref, o1_ref):
        dinv = _dinv_block(dp_ref[...])
        hw = jnp.zeros((_R, C), jnp.float32)
        for q, (a_ref, u_ref) in enumerate(
                zip((a0, a1, a2, a3), (u0, u1, u2, u3))):
            a = a_ref[...]
            hq = jnp.maximum(
                dinv * (a[0] + a[1] + u_ref[...])
                + b_ref[0, pl.ds(q * H, H)][None, :], 0.0)
            hw = hw + jnp.dot(hq, w_ref[q], preferred_element_type=jnp.float32)
        u2_ = hw * dinv
        o0_ref[...] = u2_[:, :H]
        o1_ref[...] = u2_[:, H:]
    part = pl.BlockSpec((_NC, _R, H), lambda i: (0, i, 0))
    rowb = pl.BlockSpec((_R, H), lambda i: (i, 0))
    return pl.pallas_call(
        body,
        grid=(N // _R,),
        in_specs=[part] * 4 + [rowb] * 4 + [
            pl.BlockSpec((_NC, _R, _L), lambda i: (0, i, 0)),
            pl.BlockSpec((4, H, C), lambda i: (0, 0, 0)),
            pl.BlockSpec((1, 2 * C), lambda i: (0, 0)),
        ],
        out_specs=[rowb] * 2,
        out_shape=[jax.ShapeDtypeStruct((N, H), jnp.float32)] * 2,
    )(*aggs, *us, degp, W2, b1)


def _tc_project(aggs, us, degp, b2, Wa, Wb, bfc1):
    def body(a0, a1, u0, u1, dp_ref, b2_ref, wa_ref, wb_ref, bf_ref,
             za_ref, zb_ref):
        dinv = _dinv_block(dp_ref[...])
        za = jnp.zeros((_R, C), jnp.float32)
        zb = jnp.zeros((_R, C), jnp.float32)
        for q, (a_ref, u_ref) in enumerate(zip((a0, a1), (u0, u1))):
            a = a_ref[...]
            zq = (dinv * (a[0] + a[1] + u_ref[...])
                  + b2_ref[0, pl.ds(q * H, H)][None, :])
            za = za + jnp.dot(zq, wa_ref[pl.ds(q * H, H), :],
                              preferred_element_type=jnp.float32)
            zb = zb + jnp.dot(zq, wb_ref[pl.ds(q * H, H), :],
                              preferred_element_type=jnp.float32)
        za_ref[...] = za + bf_ref[...]
        zb_ref[...] = zb
    part = pl.BlockSpec((_NC, _R, H), lambda i: (0, i, 0))
    rowb = pl.BlockSpec((_R, H), lambda i: (i, 0))
    rowc = pl.BlockSpec((_R, C), lambda i: (i, 0))
    wspec = pl.BlockSpec((C, C), lambda i: (0, 0))
    bspec = pl.BlockSpec((1, C), lambda i: (0, 0))
    return pl.pallas_call(
        body,
        grid=(N // _R,),
        in_specs=[part, part, rowb, rowb,
                  pl.BlockSpec((_NC, _R, _L), lambda i: (0, i, 0)),
                  bspec, wspec, wspec, bspec],
        out_specs=[rowc, rowc],
        out_shape=[jax.ShapeDtypeStruct((N, C), jnp.float32)] * 2,
    )(*aggs, *us, degp, b2, Wa, Wb, bfc1)


_R2 = 2000


def _tc_mlp(h1, Wfc2, bfc2, Wfc3, bfc3, Wfc4, bfc4):
    def body(h_ref, w2_ref, b2_ref, w3_ref, b3_ref, w4_ref, b4_ref, o_ref):
        t = jnp.maximum(
            jnp.dot(h_ref[...], w2_ref[...], preferred_element_type=jnp.float32)
            + b2_ref[...], 0.0)
        t = jnp.maximum(
            jnp.dot(t, w3_ref[...], preferred_element_type=jnp.float32)
            + b3_ref[...], 0.0)
        o_ref[...] = (jnp.dot(t, w4_ref[...], preferred_element_type=jnp.float32)
                      + b4_ref[...])
    full = lambda shape: pl.BlockSpec(shape, lambda i: tuple(0 for _ in shape))
    return pl.pallas_call(
        body,
        grid=(DE // _R2,),
        in_specs=[pl.BlockSpec((_R2, C), lambda i: (i, 0)),
                  full((C, 64)), full((1, 64)),
                  full((64, 32)), full((1, 32)),
                  full((32, 1)), full((1, 1))],
        out_specs=pl.BlockSpec((_R2, 1), lambda i: (i, 0)),
        out_shape=jax.ShapeDtypeStruct((DE, 1), jnp.float32),
    )(h1, Wfc2, bfc2, Wfc3, bfc3, Wfc4, bfc4)


def kernel(x, edge_index, pos_edge_index, neg_edge_index, graph_idx,
           W1, b1, W2, b2, Wfc1, bfc1, Wfc2, bfc2, Wfc3, bfc3, Wfc4, bfc4):
    ei = edge_index.astype(jnp.int32)
    src3 = ei[0].reshape(_NW, _NB, _B)
    dst3 = ei[1].reshape(_NW, _NB, _B)
    pe = pos_edge_index.astype(jnp.int32)
    ne = neg_edge_index.astype(jnp.int32)
    a3 = jnp.concatenate([pe[0], ne[0]]).reshape(_NW, _NB, _B)
    b3 = jnp.concatenate([pe[1], ne[1]]).reshape(_NW, _NB, _B)

    degp = _sc_degree(dst3)

    u1 = _tc_encode1(x, W1, degp)
    a1 = [_sc_agg(uq, src3, dst3) for uq in u1]

    u2 = _tc_encode2(a1, u1, degp, W2.reshape(4, H, C), b1.reshape(1, 2 * C))
    a2 = [_sc_agg(uq, src3, dst3) for uq in u2]

    za, zb = _tc_project(a2, u2, degp, b2.reshape(1, C),
                         Wfc1[:C], Wfc1[C:], bfc1.reshape(1, C))

    h1 = _sc_decode(za, zb, a3, b3)
    out = _tc_mlp(h1, Wfc2, bfc2.reshape(1, 64), Wfc3, bfc3.reshape(1, 32),
                  Wfc4, bfc4.reshape(1, 1))
    return out[:, 0]


# speedup vs baseline: 13.6289x; 1.7959x over previous
import functools

import jax
import jax.numpy as jnp
from jax import lax
from jax.experimental import pallas as pl
from jax.experimental.pallas import tpu as pltpu
from jax.experimental.pallas import tpu_sc as plsc

N = 10000
E = 320000
DE = 320000
C = 128
C2 = 256
H = 64

_NC, _NS, _L = 2, 16, 16
_NW = _NC * _NS
_EPT = E // _NW
_B = 80
_NB = _EPT // _B
_NP = 10240
_RP = _NP // _NS
_ZC = 128

_untiled = pltpu.CompilerParams(use_tc_tiling_on_sc=False)


def _sc_mesh():
    return plsc.VectorSubcoreMesh(
        core_axis_name="c", subcore_axis_name="s",
        num_cores=_NC, num_subcores=_NS)


def _zero_rows(buf, nrows, ncol16):
    def fill(i, _):
        for l in range(ncol16):
            buf[i, pl.ds(l * _L, _L)] = jnp.zeros((_L,), jnp.float32)
        return 0
    lax.fori_loop(0, nrows, fill, 0)


def _sc_degree(dst3):
    def body(dst_hbm, out_hbm, idx_v, ones_v, zb_v, sh):
        c = lax.axis_index("c")
        s = lax.axis_index("s")
        wid = c * _NS + s
        def fill(i, _):
            ones_v[i, pl.ds(0, _L)] = jnp.ones((_L,), jnp.float32)
            return 0
        lax.fori_loop(0, _B, fill, 0)
        _zero_rows(zb_v, _ZC, 1)
        for k in range(_RP // _ZC):
            pltpu.sync_copy(zb_v, sh.at[pl.ds(s * _RP + k * _ZC, _ZC)])
        plsc.subcore_barrier()
        pltpu.sync_copy(dst_hbm.at[wid], idx_v)
        def step(j, _):
            pltpu.sync_copy(ones_v, sh.at[idx_v.at[j]], add=True)
            return 0
        lax.fori_loop(0, _NB, step, 0)
        plsc.subcore_barrier()
        pltpu.sync_copy(sh.at[pl.ds(s * _RP, _RP)],
                        out_hbm.at[c, pl.ds(s * _RP, _RP)])

    return pl.kernel(
        body,
        out_type=jax.ShapeDtypeStruct((_NC, _NP, _L), jnp.float32),
        mesh=_sc_mesh(),
        compiler_params=_untiled,
        scratch_types=[
            pltpu.VMEM((_NB, _B), jnp.int32),
            pltpu.VMEM((_B, _L), jnp.float32),
            pltpu.VMEM((_ZC, _L), jnp.float32),
            pltpu.VMEM_SHARED((_NP, _L), jnp.float32),
        ],
    )(dst3)


def _sc_agg(u, src3, dst3):
    def body(u_hbm, src_hbm, dst_hbm, out_hbm,
             sidx, didx, rows0, rows1, zb, sh, sem0, sem1):
        c = lax.axis_index("c")
        s = lax.axis_index("s")
        wid = c * _NS + s
        _zero_rows(zb, _ZC, H // _L)
        for k in range(_RP // _ZC):
            pltpu.sync_copy(zb, sh.at[pl.ds(s * _RP + k * _ZC, _ZC)])
        plsc.subcore_barrier()
        pltpu.sync_copy(src_hbm.at[wid], sidx)
        pltpu.sync_copy(dst_hbm.at[wid], didx)

        pltpu.async_copy(u_hbm.at[sidx.at[0]], rows0, sem0)
        def step(i, _):
            j0 = 2 * i
            pltpu.async_copy(u_hbm.at[sidx.at[j0 + 1]], rows1, sem1)
            pltpu.make_async_copy(u_hbm.at[sidx.at[j0]], rows0, sem0).wait()
            pltpu.sync_copy(rows0, sh.at[didx.at[j0]], add=True)
            pltpu.async_copy(u_hbm.at[sidx.at[j0 + 2]], rows0, sem0)
            pltpu.make_async_copy(u_hbm.at[sidx.at[j0 + 1]], rows1, sem1).wait()
            pltpu.sync_copy(rows1, sh.at[didx.at[j0 + 1]], add=True)
            return 0
        lax.fori_loop(0, (_NB - 1) // 2, step, 0)
        last = _NB - 1
        pltpu.make_async_copy(u_hbm.at[sidx.at[last]], rows0, sem0).wait()
        pltpu.sync_copy(rows0, sh.at[didx.at[last]], add=True)

        plsc.subcore_barrier()
        pltpu.sync_copy(sh.at[pl.ds(s * _RP, _RP)],
                        out_hbm.at[c, pl.ds(s * _RP, _RP)])

    return pl.kernel(
        body,
        out_type=jax.ShapeDtypeStruct((_NC, _NP, H), jnp.float32),
        mesh=_sc_mesh(),
        compiler_params=_untiled,
        scratch_types=[
            pltpu.VMEM((_NB, _B), jnp.int32),
            pltpu.VMEM((_NB, _B), jnp.int32),
            pltpu.VMEM((_B, H), jnp.float32),
            pltpu.VMEM((_B, H), jnp.float32),
            pltpu.VMEM((_ZC, H), jnp.float32),
            pltpu.VMEM_SHARED((_NP, H), jnp.float32),
            pltpu.SemaphoreType.DMA,
            pltpu.SemaphoreType.DMA,
        ],
    )(u, src3, dst3)


def _sc_decode(za, zb, a3, b3):
    def body(za_hbm, zb_hbm, a_hbm, b_hbm, out_hbm,
             av, bv, ra0, rb0, ra1, rb1, sa0, sb0, sa1, sb1):
        c = lax.axis_index("c")
        s = lax.axis_index("s")
        wid = c * _NS + s
        pltpu.sync_copy(a_hbm.at[wid], av)
        pltpu.sync_copy(b_hbm.at[wid], bv)

        def comp_store(ra, rb, j):
            def comp(i, _):
                for l in range(C // _L):
                    sl = pl.ds(l * _L, _L)
                    ra[i, sl] = jnp.maximum(ra[i, sl] + rb[i, sl], 0.0)
                return 0
            lax.fori_loop(0, _B, comp, 0)
            pltpu.sync_copy(ra, out_hbm.at[pl.ds(wid * _EPT + j * _B, _B)])

        pltpu.async_copy(za_hbm.at[av.at[0]], ra0, sa0)
        pltpu.async_copy(zb_hbm.at[bv.at[0]], rb0, sb0)
        def step(i, _):
            j0 = 2 * i
            pltpu.async_copy(za_hbm.at[av.at[j0 + 1]], ra1, sa1)
            pltpu.async_copy(zb_hbm.at[bv.at[j0 + 1]], rb1, sb1)
            pltpu.make_async_copy(za_hbm.at[av.at[j0]], ra0, sa0).wait()
            pltpu.make_async_copy(zb_hbm.at[bv.at[j0]], rb0, sb0).wait()
            comp_store(ra0, rb0, j0)
            pltpu.async_copy(za_hbm.at[av.at[j0 + 2]], ra0, sa0)
            pltpu.async_copy(zb_hbm.at[bv.at[j0 + 2]], rb0, sb0)
            pltpu.make_async_copy(za_hbm.at[av.at[j0 + 1]], ra1, sa1).wait()
            pltpu.make_async_copy(zb_hbm.at[bv.at[j0 + 1]], rb1, sb1).wait()
            comp_store(ra1, rb1, j0 + 1)
            return 0
        lax.fori_loop(0, (_NB - 1) // 2, step, 0)
        last = _NB - 1
        pltpu.make_async_copy(za_hbm.at[av.at[last]], ra0, sa0).wait()
        pltpu.make_async_copy(zb_hbm.at[bv.at[last]], rb0, sb0).wait()
        comp_store(ra0, rb0, last)

    return pl.kernel(
        body,
        out_type=jax.ShapeDtypeStruct((DE, C), jnp.float32),
        mesh=_sc_mesh(),
        scratch_types=[
            pltpu.VMEM((_NB, _B), jnp.int32),
            pltpu.VMEM((_NB, _B), jnp.int32),
            pltpu.VMEM((_B, C), jnp.float32),
            pltpu.VMEM((_B, C), jnp.float32),
            pltpu.VMEM((_B, C), jnp.float32),
            pltpu.VMEM((_B, C), jnp.float32),
            pltpu.SemaphoreType.DMA,
            pltpu.SemaphoreType.DMA,
            pltpu.SemaphoreType.DMA,
            pltpu.SemaphoreType.DMA,
        ],
    )(za, zb, a3, b3)


_R = 2000


def _dinv_block(dp):
    deg = dp[0, :, 0:1] + dp[1, :, 0:1] + 1.0
    return lax.rsqrt(deg)


def _tc_encode1(x, W1, degp):
    def body(x_ref, w_ref, dp_ref, v0_ref, v1_ref, u1_ref):
        dinv = _dinv_block(dp_ref[...])
        xv = x_ref[...] * dinv
        v0_ref[...] = xv[:, :H]
        v1_ref[...] = xv[:, H:]
        xw = jnp.dot(x_ref[...], w_ref[...], preferred_element_type=jnp.float32)
        u1_ref[...] = xw * dinv
    return pl.pallas_call(
        body,
        grid=(N // _R,),
        in_specs=[
            pl.BlockSpec((_R, C), lambda i: (i, 0)),
            pl.BlockSpec((C, C2), lambda i: (0, 0)),
            pl.BlockSpec((_NC, _R, _L), lambda i: (0, i, 0)),
        ],
        out_specs=[pl.BlockSpec((_R, H), lambda i: (i, 0))] * 2
        + [pl.BlockSpec((_R, C2), lambda i: (i, 0))],
        out_shape=[jax.ShapeDtypeStruct((N, H), jnp.float32)] * 2
        + [jax.ShapeDtypeStruct((N, C2), jnp.float32)],
    )(x, W1, degp)


def _tc_encode2(vps, u1, degp, W1, W2, b1):
    def body(vp0, vp1, u1_ref, dp_ref, w1_ref, w2_ref, b_ref,
             o0_ref, o1_ref):
        dinv = _dinv_block(dp_ref[...])
        a0 = vp0[...]
        a1 = vp1[...]
        aggu = (jnp.dot(a0[0] + a0[1], w1_ref[:H, :],
                        preferred_element_type=jnp.float32)
                + jnp.dot(a1[0] + a1[1], w1_ref[H:, :],
                          preferred_element_type=jnp.float32))
        h = jnp.maximum(dinv * (aggu + u1_ref[...]) + b_ref[...], 0.0)
        u2 = dinv * jnp.dot(h, w2_ref[...], preferred_element_type=jnp.float32)
        o0_ref[...] = u2[:, :H]
        o1_ref[...] = u2[:, H:]
    part = pl.BlockSpec((_NC, _R, H), lambda i: (0, i, 0))
    rowh = pl.BlockSpec((_R, H), lambda i: (i, 0))
    return pl.pallas_call(
        body,
        grid=(N // _R,),
        in_specs=[part, part,
                  pl.BlockSpec((_R, C2), lambda i: (i, 0)),
                  pl.BlockSpec((_NC, _R, _L), lambda i: (0, i, 0)),
                  pl.BlockSpec((C, C2), lambda i: (0, 0)),
                  pl.BlockSpec((C2, C), lambda i: (0, 0)),
                  pl.BlockSpec((1, C2), lambda i: (0, 0))],
        out_specs=[rowh, rowh],
        out_shape=[jax.ShapeDtypeStruct((N, H), jnp.float32)] * 2,
    )(*vps, u1, degp, W1, W2, b1)


def _tc_project(aggs, us, degp, b2, Wa, Wb, bfc1):
    def body(a0, a1, u0, u1, dp_ref, b2_ref, wa_ref, wb_ref, bf_ref,
             za_ref, zb_ref):
        dinv = _dinv_block(dp_ref[...])
        za = jnp.zeros((_R, C), jnp.float32)
        zb = jnp.zeros((_R, C), jnp.float32)
        for q, (a_ref, u_ref) in enumerate(zip((a0, a1), (u0, u1))):
            a = a_ref[...]
            zq = (dinv * (a[0] + a[1] + u_ref[...])
                  + b2_ref[0, pl.ds(q * H, H)][None, :])
            za = za + jnp.dot(zq, wa_ref[pl.ds(q * H, H), :],
                              preferred_element_type=jnp.float32)
            zb = zb + jnp.dot(zq, wb_ref[pl.ds(q * H, H), :],
                              preferred_element_type=jnp.float32)
        za_ref[...] = za + bf_ref[...]
        zb_ref[...] = zb
    part = pl.BlockSpec((_NC, _R, H), lambda i: (0, i, 0))
    rowb = pl.BlockSpec((_R, H), lambda i: (i, 0))
    rowc = pl.BlockSpec((_R, C), lambda i: (i, 0))
    wspec = pl.BlockSpec((C, C), lambda i: (0, 0))
    bspec = pl.BlockSpec((1, C), lambda i: (0, 0))
    return pl.pallas_call(
        body,
        grid=(N // _R,),
        in_specs=[part, part, rowb, rowb,
                  pl.BlockSpec((_NC, _R, _L), lambda i: (0, i, 0)),
                  bspec, wspec, wspec, bspec],
        out_specs=[rowc, rowc],
        out_shape=[jax.ShapeDtypeStruct((N, C), jnp.float32)] * 2,
    )(*aggs, *us, degp, b2, Wa, Wb, bfc1)


_R2 = 2000


def _tc_mlp(h1, Wfc2, bfc2, Wfc3, bfc3, Wfc4, bfc4):
    def body(h_ref, w2_ref, b2_ref, w3_ref, b3_ref, w4_ref, b4_ref, o_ref):
        t = jnp.maximum(
            jnp.dot(h_ref[...], w2_ref[...], preferred_element_type=jnp.float32)
            + b2_ref[...], 0.0)
        t = jnp.maximum(
            jnp.dot(t, w3_ref[...], preferred_element_type=jnp.float32)
            + b3_ref[...], 0.0)
        o_ref[...] = (jnp.dot(t, w4_ref[...], preferred_element_type=jnp.float32)
                      + b4_ref[...])
    full = lambda shape: pl.BlockSpec(shape, lambda i: tuple(0 for _ in shape))
    return pl.pallas_call(
        body,
        grid=(DE // _R2,),
        in_specs=[pl.BlockSpec((_R2, C), lambda i: (i, 0)),
                  full((C, 64)), full((1, 64)),
                  full((64, 32)), full((1, 32)),
                  full((32, 1)), full((1, 1))],
        out_specs=pl.BlockSpec((_R2, 1), lambda i: (i, 0)),
        out_shape=jax.ShapeDtypeStruct((DE, 1), jnp.float32),
    )(h1, Wfc2, bfc2, Wfc3, bfc3, Wfc4, bfc4)


def kernel(x, edge_index, pos_edge_index, neg_edge_index, graph_idx,
           W1, b1, W2, b2, Wfc1, bfc1, Wfc2, bfc2, Wfc3, bfc3, Wfc4, bfc4):
    ei = edge_index.astype(jnp.int32)
    src3 = ei[0].reshape(_NW, _NB, _B)
    dst3 = ei[1].reshape(_NW, _NB, _B)
    pe = pos_edge_index.astype(jnp.int32)
    ne = neg_edge_index.astype(jnp.int32)
    a3 = jnp.concatenate([pe[0], ne[0]]).reshape(_NW, _NB, _B)
    b3 = jnp.concatenate([pe[1], ne[1]]).reshape(_NW, _NB, _B)

    degp = _sc_degree(dst3)

    v0, v1, u1 = _tc_encode1(x, W1, degp)
    vp = [_sc_agg(vq, src3, dst3) for vq in (v0, v1)]

    u2 = _tc_encode2(vp, u1, degp, W1, W2, b1.reshape(1, C2))
    a2 = [_sc_agg(uq, src3, dst3) for uq in u2]

    za, zb = _tc_project(a2, u2, degp, b2.reshape(1, C),
                         Wfc1[:C], Wfc1[C:], bfc1.reshape(1, C))

    h1 = _sc_decode(za, zb, a3, b3)
    out = _tc_mlp(h1, Wfc2, bfc2.reshape(1, 64), Wfc3, bfc3.reshape(1, 32),
                  Wfc4, bfc4.reshape(1, 1))
    return out[:, 0]
